# baseline (speedup 1.0000x reference)
import functools

import jax
import jax.numpy as jnp
from jax import lax
from jax.experimental import pallas as pl
from jax.experimental.pallas import tpu as pltpu
from jax.experimental.pallas import tpu_sc as plsc

_NC = 2
_NS = 16
_NW = _NC * _NS

_LROWS = 250048
_B = 4096
_F = 26
_D = 32


def _transpose_tile(tile_v, lrow_v):
    iota = lax.iota(jnp.int32, 16)

    def jj_body(jj):
        for q in range(4):
            l = jnp.full((16,), 4 * jj + q, jnp.int32)
            for h in range(2):
                d0 = h * 16
                v = plsc.load_gather(tile_v, [iota + d0, l])
                lrow_v[jj, pl.ds(q * 32 + d0, 16)] = v

    pl.loop(0, 32)(jj_body)


def _relayout_body(tT_hbm, L_hbm, tile_v, lrow_v):
    wid = lax.axis_index("s") * _NC + lax.axis_index("c")
    nw = jnp.where(wid < 4, 245, 244)

    def col_body(k):
        c = wid + 32 * k
        lane_off = pl.multiple_of(c * 128, 128)
        lrow_off = pl.multiple_of(c * 32, 32)
        pltpu.sync_copy(tT_hbm.at[:, pl.ds(lane_off, 128)], tile_v)
        _transpose_tile(tile_v, lrow_v)
        pltpu.sync_copy(lrow_v, L_hbm.at[pl.ds(lrow_off, 32)])

    pl.loop(0, nw)(col_body)


def _gather_body(xf_hbm, L_hbm, op_hbm, xblk_v, ridx_v, off_v, g_v, tbuf_v, sem):
    wid = lax.axis_index("s") * _NC + lax.axis_index("c")
    iota = lax.iota(jnp.int32, 16)
    pltpu.sync_copy(xf_hbm.at[wid], xblk_v)

    def fidx_body(f):
        for bl0 in range(0, 128, 16):
            p = iota * 26 + (bl0 * 26 + f)
            v = plsc.load_gather(xblk_v, [p >> 7, p & 127])
            ridx_v[f, pl.ds(bl0, 16)] = v >> 2
            off_v[f, pl.ds(bl0, 16)] = (v & 3) * 32

    pl.loop(0, _F)(fidx_body)

    lane_off = pl.multiple_of(wid * 128, 128)

    def f_body(f):
        pltpu.async_copy(L_hbm.at[ridx_v.at[f]], g_v, sem).wait()
        for bl0 in range(0, 128, 16):
            rowv = iota + bl0
            offv = off_v[f, pl.ds(bl0, 16)]
            for d in range(_D):
                g = plsc.load_gather(g_v, [rowv, offv + d])
                tbuf_v[d, pl.ds(bl0, 16)] = g
        pltpu.sync_copy(tbuf_v, op_hbm.at[f, :, pl.ds(lane_off, 128)])

    pl.loop(0, _F)(f_body)


@jax.jit
def kernel(x, table):
    mesh = plsc.VectorSubcoreMesh(core_axis_name="c", subcore_axis_name="s")
    params = pltpu.CompilerParams(use_tc_tiling_on_sc=True, needs_layout_passes=False)

    tT = table.T
    L = pl.kernel(
        _relayout_body,
        out_type=jax.ShapeDtypeStruct((_LROWS, 128), jnp.float32),
        mesh=mesh,
        scratch_types=[
            pltpu.VMEM((32, 128), jnp.float32),
            pltpu.VMEM((32, 128), jnp.float32),
        ],
        compiler_params=params,
    )(tT)
    tailL = table[999936:1000000].reshape(16, 128)
    L = lax.dynamic_update_slice(L, tailL, (249984, 0))

    xf3 = x.reshape(_NW, _F, 128)
    op = pl.kernel(
        _gather_body,
        out_type=jax.ShapeDtypeStruct((_F, _D, _B), jnp.float32),
        mesh=mesh,
        scratch_types=[
            pltpu.VMEM((_F, 128), jnp.int32),
            pltpu.VMEM((_F, 128), jnp.int32),
            pltpu.VMEM((_F, 128), jnp.int32),
            pltpu.VMEM((128, 128), jnp.float32),
            pltpu.VMEM((_D, 128), jnp.float32),
            pltpu.SemaphoreType.DMA,
        ],
        compiler_params=params,
    )(xf3, L)
    return jnp.transpose(op, (2, 0, 1))


# speedup vs baseline: 1.3265x; 1.3265x over previous
import functools

import jax
import jax.numpy as jnp
from jax import lax
from jax.experimental import pallas as pl
from jax.experimental.pallas import tpu as pltpu
from jax.experimental.pallas import tpu_sc as plsc

_NC = 2
_NS = 16
_NW = _NC * _NS

_LROWS = 250048
_B = 4096
_F = 26
_D = 32


def _transpose_tile(tile_r, lrow_r):
    iota = lax.iota(jnp.int32, 16)

    def jj_body(jj):
        for q in range(4):
            l = jnp.full((16,), 4 * jj + q, jnp.int32)
            for h in range(2):
                d0 = h * 16
                v = plsc.load_gather(tile_r, [iota + d0, l])
                lrow_r[jj, pl.ds(q * 32 + d0, 16)] = v

    pl.loop(0, 32)(jj_body)


def _relayout_body(tT_hbm, L_hbm, tile_v, lrow_v, si0, si1, so0, so1):
    wid = lax.axis_index("s") * _NC + lax.axis_index("c")
    nw = jnp.where(wid < 4, 245, 244)
    si = (si0, si1)
    so = (so0, so1)

    def in_desc(k, b):
        lane_off = pl.multiple_of((wid + 32 * k) * 128, 128)
        return pltpu.make_async_copy(
            tT_hbm.at[:, pl.ds(lane_off, 128)], tile_v.at[b], si[b])

    def out_desc(k, b):
        lrow_off = pl.multiple_of((wid + 32 * k) * 32, 32)
        return pltpu.make_async_copy(
            lrow_v.at[b], L_hbm.at[pl.ds(lrow_off, 32)], so[b])

    def process(k, b):
        in_desc(k, b).wait()

        @pl.when(k >= 2)
        def _wait_prev_out():
            out_desc(k - 2, b).wait()

        _transpose_tile(tile_v.at[b], lrow_v.at[b])
        out_desc(k, b).start()

        @pl.when(k + 2 < nw)
        def _next_in():
            in_desc(k + 2, b).start()

    in_desc(0, 0).start()
    in_desc(1, 1).start()

    def body(k):
        process(k, 0)

        @pl.when(k + 1 < nw)
        def _odd():
            process(k + 1, 1)

    pl.loop(0, nw, step=2)(body)
    out_desc(0, 0).wait()
    out_desc(0, 1).wait()


def _gather_body(xf_hbm, L_hbm, op_hbm, xblk_v, ridx_v, off_v, g_v, tbuf_v,
                 sg0, sg1, sv0, sv1):
    wid = lax.axis_index("s") * _NC + lax.axis_index("c")
    iota = lax.iota(jnp.int32, 16)
    sg = (sg0, sg1)
    sv = (sv0, sv1)
    pltpu.sync_copy(xf_hbm.at[wid], xblk_v)

    def fidx_body(f):
        for bl0 in range(0, 128, 16):
            p = iota * 26 + (bl0 * 26 + f)
            v = plsc.load_gather(xblk_v, [p >> 7, p & 127])
            ridx_v[f, pl.ds(bl0, 16)] = v >> 2
            off_v[f, pl.ds(bl0, 16)] = (v & 3) * 32

    pl.loop(0, _F)(fidx_body)

    lane_off = pl.multiple_of(wid * 128, 128)

    def g_desc(f, b):
        return pltpu.make_async_copy(L_hbm.at[ridx_v.at[f]], g_v.at[b], sg[b])

    def o_desc(f, b):
        return pltpu.make_async_copy(
            tbuf_v.at[b], op_hbm.at[f, :, pl.ds(lane_off, 128)], sv[b])

    def process(f, b):
        g_desc(f, b).wait()

        @pl.when(f >= 2)
        def _wait_prev_out():
            o_desc(f - 2, b).wait()

        for bl0 in range(0, 128, 16):
            rowv = iota + bl0
            offv = off_v[f, pl.ds(bl0, 16)]
            for d in range(_D):
                g = plsc.load_gather(g_v.at[b], [rowv, offv + d])
                tbuf_v[b, d, pl.ds(bl0, 16)] = g
        o_desc(f, b).start()

        @pl.when(f + 2 < _F)
        def _next_g():
            g_desc(f + 2, b).start()

    g_desc(0, 0).start()
    g_desc(1, 1).start()

    def body(f):
        process(f, 0)
        process(f + 1, 1)

    pl.loop(0, _F, step=2)(body)
    o_desc(0, 0).wait()
    o_desc(0, 1).wait()


@jax.jit
def kernel(x, table):
    mesh = plsc.VectorSubcoreMesh(core_axis_name="c", subcore_axis_name="s")
    params = pltpu.CompilerParams(use_tc_tiling_on_sc=True, needs_layout_passes=False)

    tT = table.T
    L = pl.kernel(
        _relayout_body,
        out_type=jax.ShapeDtypeStruct((_LROWS, 128), jnp.float32),
        mesh=mesh,
        scratch_types=[
            pltpu.VMEM((2, 32, 128), jnp.float32),
            pltpu.VMEM((2, 32, 128), jnp.float32),
            pltpu.SemaphoreType.DMA,
            pltpu.SemaphoreType.DMA,
            pltpu.SemaphoreType.DMA,
            pltpu.SemaphoreType.DMA,
        ],
        compiler_params=params,
    )(tT)
    tailL = table[999936:1000000].reshape(16, 128)
    L = lax.dynamic_update_slice(L, tailL, (249984, 0))

    xf3 = x.reshape(_NW, _F, 128)
    op = pl.kernel(
        _gather_body,
        out_type=jax.ShapeDtypeStruct((_F, _D, _B), jnp.float32),
        mesh=mesh,
        scratch_types=[
            pltpu.VMEM((_F, 128), jnp.int32),
            pltpu.VMEM((_F, 128), jnp.int32),
            pltpu.VMEM((_F, 128), jnp.int32),
            pltpu.VMEM((2, 128, 128), jnp.float32),
            pltpu.VMEM((2, _D, 128), jnp.float32),
            pltpu.SemaphoreType.DMA,
            pltpu.SemaphoreType.DMA,
            pltpu.SemaphoreType.DMA,
            pltpu.SemaphoreType.DMA,
        ],
        compiler_params=params,
    )(xf3, L)
    return jnp.transpose(op, (2, 0, 1))
